# baseline (speedup 1.0000x reference)
import functools

import jax
import jax.numpy as jnp
from jax import lax
from jax.experimental import pallas as pl
from jax.experimental.pallas import tpu as pltpu
from jax.experimental.pallas import tpu_sc as plsc

N = 10000
E = 320000
D = 128
H = 64
O = 128
G = 64

NSUB = 16
CH = 128
CHUNKS = E // CH
BASE_CPT = CHUNKS // NSUB
EXTRA = CHUNKS - BASE_CPT * NSUB
ROWS_PER_TILE = 632
N_PAD = ROWS_PER_TILE * NSUB
ZBLK = 128
TN = N
TROWS = TN // NSUB
IDXB = 12
NBLKS = BASE_CPT // IDXB
HC = H + 16

_mesh = plsc.VectorSubcoreMesh(core_axis_name="c", subcore_axis_name="s")


def _sc_agg_body(width, ring, xw_hbm, src_ui_hbm, dst_ui_hbm, src_iu_hbm,
                 dst_iu_hbm, *refs):
    (agg_hbm, acc_sh, xw_sh, src_v, dst_v) = refs[:5]
    rows = refs[5:5 + ring]
    gsems = refs[5 + ring:5 + 2 * ring]
    ssems = refs[5 + 2 * ring:5 + 3 * ring]

    cid = lax.axis_index("c")
    sid = lax.axis_index("s")
    zero16 = jnp.zeros((16,), jnp.float32)

    @pl.loop(0, ZBLK)
    def _(r):
        @pl.loop(0, width, step=16)
        def _(k):
            rows[0][r, pl.ds(k, 16)] = zero16

    base = sid * ROWS_PER_TILE
    @pl.loop(0, 4)
    def _(b):
        pltpu.sync_copy(rows[0], acc_sh.at[pl.ds(base + b * ZBLK, ZBLK)])
    pltpu.sync_copy(rows[0].at[pl.ds(0, ROWS_PER_TILE - 4 * ZBLK)],
                    acc_sh.at[pl.ds(base + 4 * ZBLK, ROWS_PER_TILE - 4 * ZBLK)])

    pltpu.sync_copy(xw_hbm.at[pl.ds(cid * TN + sid * TROWS, TROWS)],
                    xw_sh.at[pl.ds(sid * TROWS, TROWS)])
    plsc.subcore_barrier()

    def gstart(j, k):
        pltpu.async_copy(xw_sh.at[src_v.at[j]], rows[k], gsems[k])

    def gwait(j, k):
        pltpu.make_async_copy(xw_sh.at[src_v.at[j]], rows[k], gsems[k]).wait()

    def sstart(j, k):
        pltpu.async_copy(rows[k], acc_sh.at[dst_v.at[j]], ssems[k], add=True)

    def swait(j, k):
        pltpu.make_async_copy(rows[k], acc_sh.at[dst_v.at[j]], ssems[k]).wait()

    def run_edges(src_hbm, dst_hbm):
        start = BASE_CPT * sid + jnp.minimum(sid, EXTRA)

        @pl.loop(0, NBLKS)
        def _(b):
            blk = start + b * IDXB
            pltpu.sync_copy(src_hbm.at[pl.ds(blk, IDXB)], src_v)
            pltpu.sync_copy(dst_hbm.at[pl.ds(blk, IDXB)], dst_v)
            for i in range(ring - 1):
                gstart(i, i)

            @pl.loop(0, IDXB, step=ring)
            def _(c):
                for k in range(ring):
                    cc = c + k
                    gwait(cc, k)
                    sstart(cc, k)
                    nxt_k = (k + ring - 1) % ring

                    @pl.when(cc + ring - 1 < IDXB)
                    def _():
                        @pl.when(cc >= 1)
                        def _():
                            swait(cc - 1, nxt_k)
                        gstart(cc + ring - 1, nxt_k)

            for i in range(ring):
                swait(IDXB - ring + i, (IDXB - ring + i) % ring)

        @pl.when(sid < EXTRA)
        def _():
            e = start + BASE_CPT
            pltpu.sync_copy(src_hbm.at[pl.ds(e, 1)], src_v.at[pl.ds(0, 1)])
            pltpu.sync_copy(dst_hbm.at[pl.ds(e, 1)], dst_v.at[pl.ds(0, 1)])
            gstart(0, 0)
            gwait(0, 0)
            pltpu.sync_copy(rows[0], acc_sh.at[dst_v.at[0]], add=True)

    @pl.when(cid == 0)
    def _():
        run_edges(src_ui_hbm, dst_ui_hbm)

    @pl.when(cid == 1)
    def _():
        run_edges(src_iu_hbm, dst_iu_hbm)

    plsc.subcore_barrier()

    pltpu.sync_copy(acc_sh.at[pl.ds(base, ROWS_PER_TILE)],
                    agg_hbm.at[cid, pl.ds(base, ROWS_PER_TILE)])


def _make_sc_agg(width, ring):
    scratch = [
        pltpu.VMEM_SHARED((N_PAD, width), jnp.float32),
        pltpu.VMEM_SHARED((TN, width), jnp.float32),
        pltpu.VMEM((IDXB, CH), jnp.int32),
        pltpu.VMEM((IDXB, CH), jnp.int32),
    ]
    scratch += [pltpu.VMEM((CH, width), jnp.float32)] * ring
    scratch += [pltpu.SemaphoreType.DMA] * (2 * ring)
    return pl.kernel(
        functools.partial(_sc_agg_body, width, ring),
        out_type=jax.ShapeDtypeStruct((2, N_PAD, width), jnp.float32),
        mesh=_mesh,
        scratch_types=scratch,
        compiler_params=pltpu.CompilerParams(use_tc_tiling_on_sc=False),
    )


_sc_agg_counts = _make_sc_agg(HC, 2)
_sc_agg_plain = _make_sc_agg(H, 4)


def _dotT(x, w):
    return lax.dot_general(x, w, (((1,), (1,)), ((), ())),
                           preferred_element_type=jnp.float32)


def _pre1_body(xu, xi, wui, wiu, out):
    out[pl.ds(0, N), pl.ds(0, H)] = _dotT(xu[...], wui[...])
    out[pl.ds(N, N), pl.ds(0, H)] = _dotT(xi[...], wiu[...])
    out[pl.ds(0, 2 * N), pl.ds(H, HC - H)] = jnp.ones((2 * N, HC - H),
                                                      jnp.float32)


def _tc_pre1(x_user, x_item, wl_ui, wl_iu):
    return pl.pallas_call(
        _pre1_body,
        out_shape=jax.ShapeDtypeStruct((2 * N, HC), jnp.float32),
    )(x_user, x_item, wl_ui, wl_iu)


def _stageb_body(agg, cnt, xu, xi, wr_ui, wr_iu, b_ui, b_iu, wl2_ui, wl2_iu,
                 item1_o, user1_o, xw2_o):
    cnt_ui = jnp.maximum(cnt[0, pl.ds(0, N), :], 1.0)
    cnt_iu = jnp.maximum(cnt[1, pl.ds(0, N), :], 1.0)
    item1 = jax.nn.relu(agg[0, pl.ds(0, N), :] / cnt_ui + b_ui[...]
                        + _dotT(xi[...], wr_ui[...]))
    user1 = jax.nn.relu(agg[1, pl.ds(0, N), :] / cnt_iu + b_iu[...]
                        + _dotT(xu[...], wr_iu[...]))
    item1_o[...] = item1
    user1_o[...] = user1
    xw2_o[pl.ds(0, N), :] = _dotT(user1, wl2_ui[...])
    xw2_o[pl.ds(N, N), :] = _dotT(item1, wl2_iu[...])


def _tc_stageb(agg, cnt, x_user, x_item, wr_ui, wr_iu, b_ui, b_iu,
               wl2_ui, wl2_iu):
    return pl.pallas_call(
        _stageb_body,
        out_shape=(
            jax.ShapeDtypeStruct((N, H), jnp.float32),
            jax.ShapeDtypeStruct((N, H), jnp.float32),
            jax.ShapeDtypeStruct((2 * N, H), jnp.float32),
        ),
    )(agg, cnt, x_user, x_item, wr_ui, wr_iu, b_ui, b_iu, wl2_ui, wl2_iu)


def _stagec_body(agg, cnt, item1, user1, wr_ui, wr_iu, b_ui, b_iu,
                 batch_u, batch_i, lin_w, lin_b, out):
    cnt_ui = jnp.maximum(cnt[0, pl.ds(0, N), :], 1.0)
    cnt_iu = jnp.maximum(cnt[1, pl.ds(0, N), :], 1.0)
    item2 = jax.nn.relu(agg[0, pl.ds(0, N), :] / cnt_ui + b_ui[...]
                        + _dotT(item1[...], wr_ui[...]))
    user2 = jax.nn.relu(agg[1, pl.ds(0, N), :] / cnt_iu + b_iu[...]
                        + _dotT(user1[...], wr_iu[...]))
    gids = lax.broadcasted_iota(jnp.int32, (1, G), 1)
    oh_u = (batch_u[...] == gids).astype(jnp.float32)
    oh_i = (batch_i[...] == gids).astype(jnp.float32)
    pool_dims = (((0,), (0,)), ((), ()))
    pu = lax.dot_general(oh_u, user2, pool_dims,
                         preferred_element_type=jnp.float32)
    pi = lax.dot_general(oh_i, item2, pool_dims,
                         preferred_element_type=jnp.float32)
    cu = jnp.maximum(jnp.sum(oh_u, axis=0, keepdims=True), 1.0)
    ci = jnp.maximum(jnp.sum(oh_i, axis=0, keepdims=True), 1.0)
    g = pu / cu.T + pi / ci.T
    out[...] = _dotT(g, lin_w[...]) + lin_b[...]


def _tc_stagec(agg, cnt, item1, user1, wr_ui, wr_iu, b_ui, b_iu,
               batch_u, batch_i, lin_w, lin_b):
    return pl.pallas_call(
        _stagec_body,
        out_shape=jax.ShapeDtypeStruct((G, O), jnp.float32),
    )(agg, cnt, item1, user1, wr_ui, wr_iu, b_ui, b_iu,
      batch_u, batch_i, lin_w, lin_b)


def kernel(x_user, x_item, edge_index_ui, edge_index_iu, batch_user,
           batch_item, W_l1_ui, b1_ui, W_r1_ui, W_l1_iu, b1_iu, W_r1_iu,
           W_l2_ui, b2_ui, W_r2_ui, W_l2_iu, b2_iu, W_r2_iu, lin_W, lin_b):
    src_ui = edge_index_ui[0].reshape(CHUNKS, CH)
    dst_ui = edge_index_ui[1].reshape(CHUNKS, CH)
    src_iu = edge_index_iu[0].reshape(CHUNKS, CH)
    dst_iu = edge_index_iu[1].reshape(CHUNKS, CH)

    xw1 = _tc_pre1(x_user, x_item, W_l1_ui, W_l1_iu)
    agg1 = _sc_agg_counts(xw1, src_ui, dst_ui, src_iu, dst_iu)
    feat1 = agg1[:, :N, :H]
    cnt = agg1[:, :N, H:H + 1]
    item1, user1, xw2 = _tc_stageb(feat1, cnt, x_user, x_item, W_r1_ui,
                                   W_r1_iu, b1_ui, b1_iu, W_l2_ui, W_l2_iu)
    agg2 = _sc_agg_plain(xw2, src_ui, dst_ui, src_iu, dst_iu)
    return _tc_stagec(agg2, cnt, item1, user1, W_r2_ui, W_r2_iu, b2_ui,
                      b2_iu, batch_user.reshape(N, 1), batch_item.reshape(N, 1),
                      lin_W, lin_b)


# speedup vs baseline: 1.0289x; 1.0289x over previous
import functools

import jax
import jax.numpy as jnp
from jax import lax
from jax.experimental import pallas as pl
from jax.experimental.pallas import tpu as pltpu
from jax.experimental.pallas import tpu_sc as plsc

N = 10000
E = 320000
D = 128
H = 64
O = 128
G = 64

NSUB = 16
CH = 128
CHUNKS = E // CH
BASE_CPT = CHUNKS // NSUB
EXTRA = CHUNKS - BASE_CPT * NSUB
ROWS_PER_TILE = 632
N_PAD = ROWS_PER_TILE * NSUB
ZBLK = 128
TN = N
TROWS = TN // NSUB
IDXB = 12
NBLKS = BASE_CPT // IDXB
HC = H + 16

_mesh = plsc.VectorSubcoreMesh(core_axis_name="c", subcore_axis_name="s")


def _sc_agg_body(width, ring, xw_hbm, src_ui_hbm, dst_ui_hbm, src_iu_hbm,
                 dst_iu_hbm, *refs):
    (agg_hbm, acc_sh, xw_sh, src_v, dst_v) = refs[:5]
    rows = refs[5:5 + ring]
    gsems = refs[5 + ring:5 + 2 * ring]
    ssems = refs[5 + 2 * ring:5 + 3 * ring]

    cid = lax.axis_index("c")
    sid = lax.axis_index("s")
    zero16 = jnp.zeros((16,), jnp.float32)

    @pl.loop(0, ZBLK)
    def _(r):
        @pl.loop(0, width, step=16)
        def _(k):
            rows[0][r, pl.ds(k, 16)] = zero16

    base = sid * ROWS_PER_TILE
    @pl.loop(0, 4)
    def _(b):
        pltpu.sync_copy(rows[0], acc_sh.at[pl.ds(base + b * ZBLK, ZBLK)])
    pltpu.sync_copy(rows[0].at[pl.ds(0, ROWS_PER_TILE - 4 * ZBLK)],
                    acc_sh.at[pl.ds(base + 4 * ZBLK, ROWS_PER_TILE - 4 * ZBLK)])

    pltpu.sync_copy(xw_hbm.at[pl.ds(cid * TN + sid * TROWS, TROWS)],
                    xw_sh.at[pl.ds(sid * TROWS, TROWS)])
    plsc.subcore_barrier()

    def gstart(j, k):
        pltpu.async_copy(xw_sh.at[src_v.at[j]], rows[k], gsems[k])

    def gwait(j, k):
        pltpu.make_async_copy(xw_sh.at[src_v.at[j]], rows[k], gsems[k]).wait()

    def sstart(j, k):
        pltpu.async_copy(rows[k], acc_sh.at[dst_v.at[j]], ssems[k], add=True)

    def swait(j, k):
        pltpu.make_async_copy(rows[k], acc_sh.at[dst_v.at[j]], ssems[k]).wait()

    def run_edges(src_hbm, dst_hbm):
        start = BASE_CPT * sid + jnp.minimum(sid, EXTRA)

        @pl.loop(0, NBLKS)
        def _(b):
            blk = start + b * IDXB
            pltpu.sync_copy(src_hbm.at[pl.ds(blk, IDXB)], src_v)
            pltpu.sync_copy(dst_hbm.at[pl.ds(blk, IDXB)], dst_v)
            for i in range(ring - 1):
                gstart(i, i)

            @pl.loop(0, IDXB, step=ring)
            def _(c):
                for k in range(ring):
                    cc = c + k
                    gwait(cc, k)
                    sstart(cc, k)
                    nxt_k = (k + ring - 1) % ring

                    @pl.when(cc + ring - 1 < IDXB)
                    def _():
                        @pl.when(cc >= 1)
                        def _():
                            swait(cc - 1, nxt_k)
                        gstart(cc + ring - 1, nxt_k)

            for i in range(ring):
                swait(IDXB - ring + i, (IDXB - ring + i) % ring)

        @pl.when(sid < EXTRA)
        def _():
            e = start + BASE_CPT
            pltpu.sync_copy(src_hbm.at[pl.ds(e, 1)], src_v.at[pl.ds(0, 1)])
            pltpu.sync_copy(dst_hbm.at[pl.ds(e, 1)], dst_v.at[pl.ds(0, 1)])
            gstart(0, 0)
            gwait(0, 0)
            pltpu.sync_copy(rows[0], acc_sh.at[dst_v.at[0]], add=True)

    @pl.when(cid == 0)
    def _():
        run_edges(src_ui_hbm, dst_ui_hbm)

    @pl.when(cid == 1)
    def _():
        run_edges(src_iu_hbm, dst_iu_hbm)

    plsc.subcore_barrier()

    pltpu.sync_copy(acc_sh.at[pl.ds(base, ROWS_PER_TILE)],
                    agg_hbm.at[cid, pl.ds(base, ROWS_PER_TILE)])


def _make_sc_agg(width, ring):
    scratch = [
        pltpu.VMEM_SHARED((N_PAD, width), jnp.float32),
        pltpu.VMEM_SHARED((TN, width), jnp.float32),
        pltpu.VMEM((IDXB, CH), jnp.int32),
        pltpu.VMEM((IDXB, CH), jnp.int32),
    ]
    scratch += [pltpu.VMEM((CH, width), jnp.float32)] * ring
    scratch += [pltpu.SemaphoreType.DMA] * (2 * ring)
    return pl.kernel(
        functools.partial(_sc_agg_body, width, ring),
        out_type=jax.ShapeDtypeStruct((2, N_PAD, width), jnp.float32),
        mesh=_mesh,
        scratch_types=scratch,
        compiler_params=pltpu.CompilerParams(use_tc_tiling_on_sc=False),
    )


_sc_agg_counts = _make_sc_agg(HC, 2)
_sc_agg_plain = _make_sc_agg(H, 4)


def _dotT(x, w):
    return lax.dot_general(x, w, (((1,), (1,)), ((), ())),
                           preferred_element_type=jnp.float32)


def _pre1_body(xu, xi, wui, wiu, out):
    out[pl.ds(0, N), pl.ds(0, H)] = _dotT(xu[...], wui[...])
    out[pl.ds(N, N), pl.ds(0, H)] = _dotT(xi[...], wiu[...])
    out[pl.ds(0, 2 * N), pl.ds(H, HC - H)] = jnp.ones((2 * N, HC - H),
                                                      jnp.float32)


def _tc_pre1(x_user, x_item, wl_ui, wl_iu):
    return pl.pallas_call(
        _pre1_body,
        out_shape=jax.ShapeDtypeStruct((2 * N, HC), jnp.float32),
    )(x_user, x_item, wl_ui, wl_iu)


def _stageb_body(agg, xu, xi, wr_ui, wr_iu, b_ui, b_iu, wl2_ui, wl2_iu,
                 item1_o, user1_o, xw2_o):
    cnt_ui = jnp.maximum(agg[0, pl.ds(0, N), pl.ds(H, 1)], 1.0)
    cnt_iu = jnp.maximum(agg[1, pl.ds(0, N), pl.ds(H, 1)], 1.0)
    item1 = jax.nn.relu(agg[0, pl.ds(0, N), pl.ds(0, H)] / cnt_ui + b_ui[...]
                        + _dotT(xi[...], wr_ui[...]))
    user1 = jax.nn.relu(agg[1, pl.ds(0, N), pl.ds(0, H)] / cnt_iu + b_iu[...]
                        + _dotT(xu[...], wr_iu[...]))
    item1_o[...] = item1
    user1_o[...] = user1
    xw2_o[pl.ds(0, N), :] = _dotT(user1, wl2_ui[...])
    xw2_o[pl.ds(N, N), :] = _dotT(item1, wl2_iu[...])


def _tc_stageb(agg, x_user, x_item, wr_ui, wr_iu, b_ui, b_iu,
               wl2_ui, wl2_iu):
    return pl.pallas_call(
        _stageb_body,
        out_shape=(
            jax.ShapeDtypeStruct((N, H), jnp.float32),
            jax.ShapeDtypeStruct((N, H), jnp.float32),
            jax.ShapeDtypeStruct((2 * N, H), jnp.float32),
        ),
    )(agg, x_user, x_item, wr_ui, wr_iu, b_ui, b_iu, wl2_ui, wl2_iu)


def _stagec_body(agg, cnt, item1, user1, wr_ui, wr_iu, b_ui, b_iu,
                 batch_u, batch_i, lin_w, lin_b, out):
    cnt_ui = jnp.maximum(cnt[0, pl.ds(0, N), :], 1.0)
    cnt_iu = jnp.maximum(cnt[1, pl.ds(0, N), :], 1.0)
    item2 = jax.nn.relu(agg[0, pl.ds(0, N), :] / cnt_ui + b_ui[...]
                        + _dotT(item1[...], wr_ui[...]))
    user2 = jax.nn.relu(agg[1, pl.ds(0, N), :] / cnt_iu + b_iu[...]
                        + _dotT(user1[...], wr_iu[...]))
    gids = lax.broadcasted_iota(jnp.int32, (1, G), 1)
    oh_u = (batch_u[...] == gids).astype(jnp.float32)
    oh_i = (batch_i[...] == gids).astype(jnp.float32)
    pool_dims = (((0,), (0,)), ((), ()))
    pu = lax.dot_general(oh_u, user2, pool_dims,
                         preferred_element_type=jnp.float32)
    pi = lax.dot_general(oh_i, item2, pool_dims,
                         preferred_element_type=jnp.float32)
    cu = jnp.maximum(jnp.sum(oh_u, axis=0, keepdims=True), 1.0)
    ci = jnp.maximum(jnp.sum(oh_i, axis=0, keepdims=True), 1.0)
    g = pu / cu.T + pi / ci.T
    out[...] = _dotT(g, lin_w[...]) + lin_b[...]


def _tc_stagec(agg, cnt, item1, user1, wr_ui, wr_iu, b_ui, b_iu,
               batch_u, batch_i, lin_w, lin_b):
    return pl.pallas_call(
        _stagec_body,
        out_shape=jax.ShapeDtypeStruct((G, O), jnp.float32),
    )(agg, cnt, item1, user1, wr_ui, wr_iu, b_ui, b_iu,
      batch_u, batch_i, lin_w, lin_b)


def kernel(x_user, x_item, edge_index_ui, edge_index_iu, batch_user,
           batch_item, W_l1_ui, b1_ui, W_r1_ui, W_l1_iu, b1_iu, W_r1_iu,
           W_l2_ui, b2_ui, W_r2_ui, W_l2_iu, b2_iu, W_r2_iu, lin_W, lin_b):
    src_ui = edge_index_ui[0].reshape(CHUNKS, CH)
    dst_ui = edge_index_ui[1].reshape(CHUNKS, CH)
    src_iu = edge_index_iu[0].reshape(CHUNKS, CH)
    dst_iu = edge_index_iu[1].reshape(CHUNKS, CH)

    xw1 = _tc_pre1(x_user, x_item, W_l1_ui, W_l1_iu)
    agg1 = _sc_agg_counts(xw1, src_ui, dst_ui, src_iu, dst_iu)
    cnt = agg1[:, :N, H:H + 1]
    item1, user1, xw2 = _tc_stageb(agg1, x_user, x_item, W_r1_ui,
                                   W_r1_iu, b1_ui, b1_iu, W_l2_ui, W_l2_iu)
    agg2 = _sc_agg_plain(xw2, src_ui, dst_ui, src_iu, dst_iu)
    return _tc_stagec(agg2, cnt, item1, user1, W_r2_ui, W_r2_iu, b2_ui,
                      b2_iu, batch_user.reshape(N, 1), batch_item.reshape(N, 1),
                      lin_W, lin_b)
